# baseline (speedup 1.0000x reference)
import jax
import jax.numpy as jnp
from jax.experimental import pallas as pl


def _add_body(t_ref, k_ref, o_ref):
    o_ref[...] = t_ref[...] + k_ref[...]


def kernel(src, mask, pos_embed, tgt, tgt_mask, class_feature, kp_template):
    S, B, D = tgt.shape
    LANES = 1024
    rows = S * B * D // LANES
    t2 = tgt.reshape(rows, LANES)
    kp_row = jnp.tile(kp_template, LANES // D).reshape(1, LANES)

    BLK = 512
    grid = (rows // BLK,)
    out = pl.pallas_call(
        _add_body,
        grid=grid,
        in_specs=[
            pl.BlockSpec((BLK, LANES), lambda i: (i, 0)),
            pl.BlockSpec((1, LANES), lambda i: (0, 0)),
        ],
        out_specs=pl.BlockSpec((BLK, LANES), lambda i: (i, 0)),
        out_shape=jax.ShapeDtypeStruct((rows, LANES), tgt.dtype),
    )(t2, kp_row)
    return out.reshape(S, B, D)


# speedup vs baseline: 1.8390x; 1.8390x over previous
import jax
import jax.numpy as jnp
from jax.experimental import pallas as pl


def _add_body(t_ref, k_ref, o_ref):
    o_ref[...] = t_ref[...] + k_ref[...]


def kernel(src, mask, pos_embed, tgt, tgt_mask, class_feature, kp_template):
    S, B, D = tgt.shape
    rows = S * B
    t2 = tgt.reshape(rows, D)
    kp_row = kp_template.reshape(1, D)

    BLK = 8192
    grid = (rows // BLK,)
    out = pl.pallas_call(
        _add_body,
        grid=grid,
        in_specs=[
            pl.BlockSpec((BLK, D), lambda i: (i, 0)),
            pl.BlockSpec((1, D), lambda i: (0, 0)),
        ],
        out_specs=pl.BlockSpec((BLK, D), lambda i: (i, 0)),
        out_shape=jax.ShapeDtypeStruct((rows, D), tgt.dtype),
    )(t2, kp_row)
    return out.reshape(S, B, D)
